# baseline (speedup 1.0000x reference)
import functools

import jax
import jax.numpy as jnp
from jax import lax
from jax.experimental import pallas as pl
from jax.experimental.pallas import tpu as pltpu
from jax.experimental.pallas import tpu_sc as plsc

K = 128
CH = 128
NBUF = 6
PF = 4
W = 128


def _make_kernel(N, n_chunks, per_w, NC):
    mesh = plsc.VectorSubcoreMesh(core_axis_name="c", subcore_axis_name="s")

    @functools.partial(
        pl.kernel,
        mesh=mesh,
        out_type=jax.ShapeDtypeStruct((N, W), jnp.float32),
        compiler_params=pltpu.CompilerParams(use_tc_tiling_on_sc=True),
        scratch_types=[
            pltpu.VMEM((per_w,), jnp.int32),
            pltpu.VMEM((NBUF, CH, W), jnp.float32),
            [pltpu.SemaphoreType.DMA] * NBUF,
            [pltpu.SemaphoreType.DMA] * NBUF,
        ],
    )
    def emb_kernel(idx_hbm, table_hbm, out_hbm, idx_v, rows_v, gsems, osems):
        wid = lax.axis_index("s") * NC + lax.axis_index("c")
        row_base = wid * per_w

        pltpu.sync_copy(
            idx_hbm.at[pl.ds(pl.multiple_of(row_base, CH), per_w)], idx_v
        )

        def fire_gathers(c, b):
            pltpu.async_copy(
                table_hbm.at[idx_v.at[pl.ds(c * CH, K)]],
                rows_v.at[b],
                gsems[b],
            )

        def wait_gathers(c, b):
            pltpu.make_async_copy(
                table_hbm.at[idx_v.at[pl.ds(c * CH, K)]],
                rows_v.at[b],
                gsems[b],
            ).wait()

        def fire_out(c, b):
            off = pl.multiple_of(row_base + c * CH, CH)
            pltpu.async_copy(rows_v.at[b], out_hbm.at[pl.ds(off, CH)], osems[b])

        def wait_out(c, b):
            off = pl.multiple_of(row_base + c * CH, CH)
            pltpu.make_async_copy(
                rows_v.at[b], out_hbm.at[pl.ds(off, CH)], osems[b]
            ).wait()

        for c in range(PF):
            fire_gathers(c, c % NBUF)
        for c in range(PF):
            cg = c + PF
            if cg >= NBUF:
                wait_out(cg - NBUF, cg % NBUF)
            fire_gathers(cg, cg % NBUF)
            wait_gathers(c, c % NBUF)
            fire_out(c, c % NBUF)

        n_steady = n_chunks - 2 * PF
        assert n_steady % NBUF == 0
        n_groups = n_steady // NBUF

        def group(it, carry):
            c0 = PF + it * NBUF
            for k in range(NBUF):
                c = c0 + k
                b = (PF + k) % NBUF
                b2 = (PF + k + PF) % NBUF
                wait_out(c + PF - NBUF, b2)
                fire_gathers(c + PF, b2)
                wait_gathers(c, b)
                fire_out(c, b)
            return carry

        lax.fori_loop(0, n_groups, group, 0)

        for c in range(n_chunks - PF, n_chunks):
            b = c % NBUF
            wait_gathers(c, b)
            fire_out(c, b)
        for c in range(n_chunks - NBUF, n_chunks):
            wait_out(c, c % NBUF)

    return emb_kernel


def kernel(indexseq, table):
    B, S = indexseq.shape
    V, D = table.shape
    N = B * S
    info = plsc.get_sparse_core_info()
    NW = info.num_cores * info.num_subcores
    per_w = N // NW
    n_chunks = per_w // CH
    tab128 = jnp.pad(table, ((0, 0), (0, W - D)))
    idx1d = indexseq.T.reshape(N)
    out = _make_kernel(N, n_chunks, per_w, info.num_cores)(idx1d, tab128)
    return out[:, :D].reshape(S, B, D)


# speedup vs baseline: 1.0028x; 1.0028x over previous
import functools

import jax
import jax.numpy as jnp
from jax import lax
from jax.experimental import pallas as pl
from jax.experimental.pallas import tpu as pltpu
from jax.experimental.pallas import tpu_sc as plsc

K = 128
CH = 128
NBUF = 6
PF = 4
W = 128


def _make_kernel(N, n_chunks, per_w, NC):
    mesh = plsc.VectorSubcoreMesh(core_axis_name="c", subcore_axis_name="s")

    @functools.partial(
        pl.kernel,
        mesh=mesh,
        out_type=jax.ShapeDtypeStruct((N, W), jnp.float32),
        compiler_params=pltpu.CompilerParams(use_tc_tiling_on_sc=True),
        scratch_types=[
            pltpu.VMEM((per_w,), jnp.int32),
            pltpu.VMEM((NBUF, CH, W), jnp.float32),
            [pltpu.SemaphoreType.DMA] * NBUF,
            [pltpu.SemaphoreType.DMA] * NBUF,
        ],
    )
    def emb_kernel(idx_hbm, table_hbm, out_hbm, idx_v, rows_v, gsems, osems):
        wid = lax.axis_index("s") * NC + lax.axis_index("c")
        row_base = wid * per_w

        pltpu.sync_copy(
            idx_hbm.at[pl.ds(pl.multiple_of(row_base, CH), per_w)], idx_v
        )

        def fire_gathers(c, b):
            pltpu.async_copy(
                table_hbm.at[idx_v.at[pl.ds(c * CH, K)]],
                rows_v.at[b],
                gsems[b],
            )

        def wait_gathers(c, b):
            pltpu.make_async_copy(
                table_hbm.at[idx_v.at[pl.ds(c * CH, K)]],
                rows_v.at[b],
                gsems[b],
            ).wait()

        def fire_out(c, b):
            off = pl.multiple_of(row_base + c * CH, CH)
            pltpu.async_copy(rows_v.at[b], out_hbm.at[pl.ds(off, CH)], osems[b])

        def wait_out(c, b):
            off = pl.multiple_of(row_base + c * CH, CH)
            pltpu.make_async_copy(
                rows_v.at[b], out_hbm.at[pl.ds(off, CH)], osems[b]
            ).wait()

        for c in range(PF):
            fire_gathers(c, c % NBUF)
        for c in range(PF):
            cg = c + PF
            if cg >= NBUF:
                wait_out(cg - NBUF, cg % NBUF)
            fire_gathers(cg, cg % NBUF)
            wait_gathers(c, c % NBUF)
            fire_out(c, c % NBUF)

        n_steady = n_chunks - 2 * PF
        assert n_steady % NBUF == 0
        n_groups = n_steady // NBUF

        def group(it, carry):
            c0 = PF + it * NBUF
            for k in range(NBUF):
                c = c0 + k
                b = (PF + k) % NBUF
                b2 = (PF + k + PF) % NBUF
                wait_out(c + PF - NBUF, b2)
                fire_gathers(c + PF, b2)
                wait_gathers(c, b)
                fire_out(c, b)
            return carry

        lax.fori_loop(0, n_groups, group, 0)

        for c in range(n_chunks - PF, n_chunks):
            b = c % NBUF
            wait_gathers(c, b)
            fire_out(c, b)
        for c in range(n_chunks - NBUF, n_chunks):
            wait_out(c, c % NBUF)

    return emb_kernel


def kernel(indexseq, table):
    B, S = indexseq.shape
    V, D = table.shape
    N = B * S
    info = plsc.get_sparse_core_info()
    NW = info.num_cores * info.num_subcores
    per_w = N // NW
    n_chunks = per_w // CH
    tab128 = jnp.pad(table.T, ((0, W - D), (0, 0))).T
    idx1d = indexseq.T.reshape(N)
    out = _make_kernel(N, n_chunks, per_w, info.num_cores)(idx1d, tab128)
    return out[:, :D].reshape(S, B, D)
